# baseline (speedup 1.0000x reference)
import functools

import jax
import jax.numpy as jnp
from jax import lax
from jax.experimental import pallas as pl
from jax.experimental.pallas import tpu as pltpu
from jax.experimental.pallas import tpu_sc as plsc

_N = 10000
_E = 320000
_EMB = 128
_HID = 128
_NCLS = 10
_NGRAPH = 64

_NC = 2
_NS = 16
_NW = _NC * _NS

_NP = 10240
_RPW = _NP // _NW
_GCH = 80
_NGCH = _RPW // _GCH
_EP = 327680
_ET = _EP // _NW
_ECH = 128
_NCHUNK = _ET // _ECH
_HCH = _NCHUNK // 2
_HPAIR = _HCH // 2
_EDW = _E // _NW
_SLAB = _NP // _NS

_BR = 1024
_NBLK = _NP // _BR

_sc_mesh = plsc.VectorSubcoreMesh(core_axis_name="c", subcore_axis_name="s")


@functools.partial(
    pl.kernel,
    out_type=[
        jax.ShapeDtypeStruct((_NP, _EMB), jnp.float32),
        jax.ShapeDtypeStruct((_NW, _NP), jnp.float32),
    ],
    mesh=_sc_mesh,
    scratch_types=[
        pltpu.VMEM((_NGCH, _GCH), jnp.int32),
        pltpu.VMEM((_GCH, _EMB), jnp.float32),
        pltpu.VMEM((_EDW,), jnp.int32),
        pltpu.VMEM((_NP,), jnp.float32),
        pltpu.SemaphoreType.DMA,
    ],
    compiler_params=pltpu.CompilerParams(needs_layout_passes=False),
)
def _sc_gather_deg(emb_hbm, xp_hbm, dstw_hbm, h0_hbm, cnt_hbm,
                   xbuf, rowbuf, dstbuf, cntbuf, sem):
    cid = lax.axis_index("c")
    sid = lax.axis_index("s")
    wid = sid * _NC + cid

    pltpu.sync_copy(xp_hbm.at[wid], xbuf)
    for j in range(_NGCH):
        pltpu.async_copy(emb_hbm.at[xbuf.at[j]], rowbuf, sem).wait()
        pltpu.sync_copy(rowbuf, h0_hbm.at[pl.ds(wid * _RPW + j * _GCH, _GCH)])

    pltpu.sync_copy(dstw_hbm.at[wid], dstbuf)
    zero16 = jnp.zeros((16,), jnp.float32)

    def zbody(i, carry):
        cntbuf[pl.ds(i * 16, 16)] = zero16
        return carry

    lax.fori_loop(0, _NP // 16, zbody, 0)

    ones16 = jnp.full((16,), 1.0, jnp.float32)

    def cbody(i, carry):
        idxv = dstbuf[pl.ds(i * 16, 16)]
        plsc.addupdate_scatter(cntbuf, [idxv], ones16)
        return carry

    lax.fori_loop(0, _EDW // 16, cbody, 0)
    pltpu.sync_copy(cntbuf, cnt_hbm.at[wid])


@functools.partial(
    pl.kernel,
    out_type=jax.ShapeDtypeStruct((_NC, _NP, _HID), jnp.float32),
    mesh=_sc_mesh,
    scratch_types=[
        pltpu.VMEM((_NCHUNK, _ECH), jnp.int32),
        pltpu.VMEM((_NCHUNK, _ECH), jnp.int32),
        pltpu.VMEM((_ECH, _HID), jnp.float32),
        pltpu.VMEM_SHARED((_NP, _HID), jnp.float32),
        pltpu.SemaphoreType.DMA,
    ],
)
def _sc_scatter(g_hbm, srcr_hbm, dstr_hbm, zeros_hbm, out_hbm,
                sbuf, dbuf, rowbuf, acc, sem):
    cid = lax.axis_index("c")
    sid = lax.axis_index("s")
    wid = sid * _NC + cid

    pltpu.sync_copy(zeros_hbm, acc.at[pl.ds(sid * _SLAB, _SLAB)])
    pltpu.sync_copy(srcr_hbm.at[wid], sbuf)
    pltpu.sync_copy(dstr_hbm.at[wid], dbuf)
    plsc.subcore_barrier()

    def body(j, carry):
        pltpu.async_copy(g_hbm.at[sbuf.at[j]], rowbuf, sem).wait()
        pltpu.sync_copy(rowbuf, acc.at[dbuf.at[j]], add=True)
        return carry

    lax.fori_loop(0, _NCHUNK, body, 0)
    plsc.subcore_barrier()
    pltpu.sync_copy(acc.at[pl.ds(sid * _SLAB, _SLAB)],
                    out_hbm.at[cid, pl.ds(sid * _SLAB, _SLAB)])


def _tc_prep1_body(h0_ref, w_ref, cnt_ref, ones_ref, g_ref, dis_ref):
    deg = lax.dot_general(cnt_ref[...], ones_ref[...], (((0,), (0,)), ((), ())),
                          preferred_element_type=jnp.float32)
    dis = lax.rsqrt(deg + 1.0)
    g_ref[...] = dis * jnp.dot(h0_ref[...], w_ref[...],
                               preferred_element_type=jnp.float32)
    dis_ref[...] = dis


_tc_prep1 = pl.pallas_call(
    _tc_prep1_body,
    grid=(_NBLK,),
    in_specs=[
        pl.BlockSpec((_BR, _EMB), lambda i: (i, 0)),
        pl.BlockSpec((_EMB, _HID), lambda i: (0, 0)),
        pl.BlockSpec((_NW, _BR), lambda i: (0, i)),
        pl.BlockSpec((_NW, 1), lambda i: (0, 0)),
    ],
    out_specs=[
        pl.BlockSpec((_BR, _HID), lambda i: (i, 0)),
        pl.BlockSpec((_BR, 1), lambda i: (i, 0)),
    ],
    out_shape=[
        jax.ShapeDtypeStruct((_NP, _HID), jnp.float32),
        jax.ShapeDtypeStruct((_NP, 1), jnp.float32),
    ],
)


def _tc_mid_body(p0_ref, p1_ref, g_ref, dis_ref, b_ref, w_ref, g2_ref):
    dis = dis_ref[...]
    h = jnp.maximum(dis * (p0_ref[...] + p1_ref[...] + g_ref[...]) + b_ref[...],
                    0.0)
    g2_ref[...] = dis * jnp.dot(h, w_ref[...],
                                preferred_element_type=jnp.float32)


_tc_mid = pl.pallas_call(
    _tc_mid_body,
    grid=(_NBLK,),
    in_specs=[
        pl.BlockSpec((_BR, _HID), lambda i: (i, 0)),
        pl.BlockSpec((_BR, _HID), lambda i: (i, 0)),
        pl.BlockSpec((_BR, _HID), lambda i: (i, 0)),
        pl.BlockSpec((_BR, 1), lambda i: (i, 0)),
        pl.BlockSpec((1, _HID), lambda i: (0, 0)),
        pl.BlockSpec((_HID, _HID), lambda i: (0, 0)),
    ],
    out_specs=pl.BlockSpec((_BR, _HID), lambda i: (i, 0)),
    out_shape=jax.ShapeDtypeStruct((_NP, _HID), jnp.float32),
)


def _tc_final_body(p0_ref, p1_ref, g_ref, dis_ref, b_ref, batch_ref,
                   linw_ref, linb_ref, out_ref, pacc, cacc):
    i = pl.program_id(0)
    dis = dis_ref[...]
    h = jnp.maximum(dis * (p0_ref[...] + p1_ref[...] + g_ref[...]) + b_ref[...],
                    0.0)
    gids = lax.broadcasted_iota(jnp.int32, (1, _NGRAPH), 1)
    onehot = (batch_ref[...] == gids).astype(jnp.float32)
    ps = lax.dot_general(onehot, h, (((0,), (0,)), ((), ())),
                         preferred_element_type=jnp.float32)
    cs = lax.dot_general(onehot, jnp.ones((_BR, 1), jnp.float32),
                         (((0,), (0,)), ((), ())),
                         preferred_element_type=jnp.float32)

    @pl.when(i == 0)
    def _():
        pacc[...] = ps
        cacc[...] = cs

    @pl.when(i != 0)
    def _():
        pacc[...] += ps
        cacc[...] += cs

    @pl.when(i == _NBLK - 1)
    def _():
        pooled = pacc[...] / jnp.maximum(cacc[...], 1.0)
        out_ref[...] = (jnp.dot(pooled, linw_ref[...],
                                preferred_element_type=jnp.float32)
                        + linb_ref[...])


_tc_final = pl.pallas_call(
    _tc_final_body,
    grid=(_NBLK,),
    in_specs=[
        pl.BlockSpec((_BR, _HID), lambda i: (i, 0)),
        pl.BlockSpec((_BR, _HID), lambda i: (i, 0)),
        pl.BlockSpec((_BR, _HID), lambda i: (i, 0)),
        pl.BlockSpec((_BR, 1), lambda i: (i, 0)),
        pl.BlockSpec((1, _HID), lambda i: (0, 0)),
        pl.BlockSpec((_BR, 1), lambda i: (i, 0)),
        pl.BlockSpec((_HID, _NCLS), lambda i: (0, 0)),
        pl.BlockSpec((1, _NCLS), lambda i: (0, 0)),
    ],
    out_specs=pl.BlockSpec((_NGRAPH, _NCLS), lambda i: (0, 0)),
    out_shape=jax.ShapeDtypeStruct((_NGRAPH, _NCLS), jnp.float32),
    scratch_shapes=[
        pltpu.VMEM((_NGRAPH, _HID), jnp.float32),
        pltpu.VMEM((_NGRAPH, 1), jnp.float32),
    ],
)


def kernel(x, edge_index, batch, emb, W1, b1, W2, b2, linW, linb):
    x = x.astype(jnp.int32)
    src = edge_index[0].astype(jnp.int32)
    dst = edge_index[1].astype(jnp.int32)

    xp = jnp.concatenate([x, jnp.zeros((_NP - _N,), jnp.int32)])
    xp = xp.reshape(_NW, _NGCH, _GCH)
    dstw = dst.reshape(_NW, _EDW)
    srcr = jnp.concatenate([src, jnp.zeros((_EP - _E,), jnp.int32)])
    srcr = srcr.reshape(_NW, _NCHUNK, _ECH)
    dstr = jnp.concatenate([dst, jnp.full((_EP - _E,), _N, jnp.int32)])
    dstr = dstr.reshape(_NW, _NCHUNK, _ECH)
    batch2d = jnp.concatenate(
        [batch.astype(jnp.int32), jnp.full((_NP - _N,), _NGRAPH, jnp.int32)]
    ).reshape(_NP, 1)
    zeros_slab = jnp.zeros((_SLAB, _HID), jnp.float32)
    ones_nw = jnp.ones((_NW, 1), jnp.float32)

    h0, cnt = _sc_gather_deg(emb, xp, dstw)
    g1, dis = _tc_prep1(h0, W1, cnt, ones_nw)
    p1 = _sc_scatter(g1, srcr, dstr, zeros_slab)
    g2 = _tc_mid(p1[0], p1[1], g1, dis, b1.reshape(1, _HID), W2)
    p2 = _sc_scatter(g2, srcr, dstr, zeros_slab)
    out = _tc_final(p2[0], p2[1], g2, dis, b2.reshape(1, _HID), batch2d,
                    linW, linb.reshape(1, _NCLS))
    return out


# speedup vs baseline: 1.0008x; 1.0008x over previous
import functools

import jax
import jax.numpy as jnp
from jax import lax
from jax.experimental import pallas as pl
from jax.experimental.pallas import tpu as pltpu
from jax.experimental.pallas import tpu_sc as plsc

_N = 10000
_E = 320000
_EMB = 128
_HID = 128
_NCLS = 10
_NGRAPH = 64

_NC = 2
_NS = 16
_NW = _NC * _NS

_NP = 10240
_RPW = _NP // _NW
_GCH = 80
_NGCH = _RPW // _GCH
_EP = 327680
_ET = _EP // _NW
_ECH = 128
_NCHUNK = _ET // _ECH
_HCH = _NCHUNK // 2
_HPAIR = _HCH // 2
_EDW = _E // _NW
_SLAB = _NP // _NS

_BR = 1024
_NBLK = _NP // _BR

_sc_mesh = plsc.VectorSubcoreMesh(core_axis_name="c", subcore_axis_name="s")


@functools.partial(
    pl.kernel,
    out_type=[
        jax.ShapeDtypeStruct((_NP, _EMB), jnp.float32),
        jax.ShapeDtypeStruct((_NW, _NP), jnp.float32),
    ],
    mesh=_sc_mesh,
    scratch_types=[
        pltpu.VMEM((_NGCH, _GCH), jnp.int32),
        pltpu.VMEM((_GCH, _EMB), jnp.float32),
        pltpu.VMEM((_EDW,), jnp.int32),
        pltpu.VMEM((_NP,), jnp.float32),
        pltpu.SemaphoreType.DMA,
    ],
    compiler_params=pltpu.CompilerParams(needs_layout_passes=False),
)
def _sc_gather_deg(emb_hbm, xp_hbm, dstw_hbm, h0_hbm, cnt_hbm,
                   xbuf, rowbuf, dstbuf, cntbuf, sem):
    cid = lax.axis_index("c")
    sid = lax.axis_index("s")
    wid = sid * _NC + cid

    pltpu.sync_copy(xp_hbm.at[wid], xbuf)
    for j in range(_NGCH):
        pltpu.async_copy(emb_hbm.at[xbuf.at[j]], rowbuf, sem).wait()
        pltpu.sync_copy(rowbuf, h0_hbm.at[pl.ds(wid * _RPW + j * _GCH, _GCH)])

    pltpu.sync_copy(dstw_hbm.at[wid], dstbuf)
    zero16 = jnp.zeros((16,), jnp.float32)

    def zbody(i, carry):
        cntbuf[pl.ds(i * 16, 16)] = zero16
        return carry

    lax.fori_loop(0, _NP // 16, zbody, 0)

    ones16 = jnp.full((16,), 1.0, jnp.float32)

    def cbody(i, carry):
        idxv = dstbuf[pl.ds(i * 16, 16)]
        plsc.addupdate_scatter(cntbuf, [idxv], ones16)
        return carry

    lax.fori_loop(0, _EDW // 16, cbody, 0)
    pltpu.sync_copy(cntbuf, cnt_hbm.at[wid])


@functools.partial(
    pl.kernel,
    out_type=jax.ShapeDtypeStruct((_NC, _NP, _HID), jnp.float32),
    mesh=_sc_mesh,
    scratch_types=[
        pltpu.VMEM((_NCHUNK, _ECH), jnp.int32),
        pltpu.VMEM((_NCHUNK, _ECH), jnp.int32),
        pltpu.VMEM((_ECH, _HID), jnp.float32),
        pltpu.VMEM_SHARED((_NP, _HID), jnp.float32),
        pltpu.SemaphoreType.DMA,
    ],
)
def _sc_scatter(g_hbm, srcr_hbm, dstr_hbm, zeros_hbm, out_hbm,
                sbuf, dbuf, rowbuf, acc, sem):
    cid = lax.axis_index("c")
    sid = lax.axis_index("s")
    wid = sid * _NC + cid

    pltpu.sync_copy(zeros_hbm, acc.at[pl.ds(sid * _SLAB, _SLAB)])
    pltpu.sync_copy(srcr_hbm.at[wid], sbuf)
    pltpu.sync_copy(dstr_hbm.at[wid], dbuf)
    plsc.subcore_barrier()

    def body(j, carry):
        pltpu.async_copy(g_hbm.at[sbuf.at[j]], rowbuf, sem).wait()
        pltpu.sync_copy(rowbuf, acc.at[dbuf.at[j]], add=True)
        return carry

    lax.fori_loop(0, _NCHUNK, body, 0)
    plsc.subcore_barrier()
    pltpu.sync_copy(acc.at[pl.ds(sid * _SLAB, _SLAB)],
                    out_hbm.at[cid, pl.ds(sid * _SLAB, _SLAB)])


def _tc_prep1_body(h0_ref, w_ref, cnt_ref, ones_ref, g_ref, dis_ref):
    deg = lax.dot_general(cnt_ref[...], ones_ref[...], (((0,), (0,)), ((), ())),
                          preferred_element_type=jnp.float32)
    dis = lax.rsqrt(deg + 1.0)
    g_ref[...] = dis * jnp.dot(h0_ref[...], w_ref[...],
                               preferred_element_type=jnp.float32)
    dis_ref[...] = dis


_tc_prep1 = pl.pallas_call(
    _tc_prep1_body,
    grid=(_NBLK,),
    in_specs=[
        pl.BlockSpec((_BR, _EMB), lambda i: (i, 0)),
        pl.BlockSpec((_EMB, _HID), lambda i: (0, 0)),
        pl.BlockSpec((_NW, _BR), lambda i: (0, i)),
        pl.BlockSpec((_NW, 1), lambda i: (0, 0)),
    ],
    out_specs=[
        pl.BlockSpec((_BR, _HID), lambda i: (i, 0)),
        pl.BlockSpec((_BR, 1), lambda i: (i, 0)),
    ],
    out_shape=[
        jax.ShapeDtypeStruct((_NP, _HID), jnp.float32),
        jax.ShapeDtypeStruct((_NP, 1), jnp.float32),
    ],
)


def _tc_mid_body(p0_ref, p1_ref, g_ref, dis_ref, b_ref, w_ref, g2_ref):
    dis = dis_ref[...]
    h = jnp.maximum(dis * (p0_ref[...] + p1_ref[...] + g_ref[...]) + b_ref[...],
                    0.0)
    g2_ref[...] = dis * jnp.dot(h, w_ref[...],
                                preferred_element_type=jnp.float32)


_tc_mid = pl.pallas_call(
    _tc_mid_body,
    grid=(_NBLK,),
    in_specs=[
        pl.BlockSpec((_BR, _HID), lambda i: (i, 0)),
        pl.BlockSpec((_BR, _HID), lambda i: (i, 0)),
        pl.BlockSpec((_BR, _HID), lambda i: (i, 0)),
        pl.BlockSpec((_BR, 1), lambda i: (i, 0)),
        pl.BlockSpec((1, _HID), lambda i: (0, 0)),
        pl.BlockSpec((_HID, _HID), lambda i: (0, 0)),
    ],
    out_specs=pl.BlockSpec((_BR, _HID), lambda i: (i, 0)),
    out_shape=jax.ShapeDtypeStruct((_NP, _HID), jnp.float32),
)


def _tc_final_body(p0_ref, p1_ref, g_ref, dis_ref, b_ref, batch_ref,
                   linw_ref, linb_ref, out_ref, pacc, cacc):
    i = pl.program_id(0)
    dis = dis_ref[...]
    h = jnp.maximum(dis * (p0_ref[...] + p1_ref[...] + g_ref[...]) + b_ref[...],
                    0.0)
    gids = lax.broadcasted_iota(jnp.int32, (1, _NGRAPH), 1)
    onehot = (batch_ref[...] == gids).astype(jnp.float32)
    ps = lax.dot_general(onehot, h, (((0,), (0,)), ((), ())),
                         preferred_element_type=jnp.float32)
    cs = lax.dot_general(onehot, jnp.ones((_BR, 1), jnp.float32),
                         (((0,), (0,)), ((), ())),
                         preferred_element_type=jnp.float32)

    @pl.when(i == 0)
    def _():
        pacc[...] = ps
        cacc[...] = cs

    @pl.when(i != 0)
    def _():
        pacc[...] += ps
        cacc[...] += cs

    @pl.when(i == _NBLK - 1)
    def _():
        pooled = pacc[...] / jnp.maximum(cacc[...], 1.0)
        out_ref[...] = (jnp.dot(pooled, linw_ref[...],
                                preferred_element_type=jnp.float32)
                        + linb_ref[...])


_tc_final = pl.pallas_call(
    _tc_final_body,
    grid=(_NBLK,),
    in_specs=[
        pl.BlockSpec((_BR, _HID), lambda i: (i, 0)),
        pl.BlockSpec((_BR, _HID), lambda i: (i, 0)),
        pl.BlockSpec((_BR, _HID), lambda i: (i, 0)),
        pl.BlockSpec((_BR, 1), lambda i: (i, 0)),
        pl.BlockSpec((1, _HID), lambda i: (0, 0)),
        pl.BlockSpec((_BR, 1), lambda i: (i, 0)),
        pl.BlockSpec((_HID, _NCLS), lambda i: (0, 0)),
        pl.BlockSpec((1, _NCLS), lambda i: (0, 0)),
    ],
    out_specs=pl.BlockSpec((_NGRAPH, _NCLS), lambda i: (0, 0)),
    out_shape=jax.ShapeDtypeStruct((_NGRAPH, _NCLS), jnp.float32),
    scratch_shapes=[
        pltpu.VMEM((_NGRAPH, _HID), jnp.float32),
        pltpu.VMEM((_NGRAPH, 1), jnp.float32),
    ],
)


def kernel(x, edge_index, batch, emb, W1, b1, W2, b2, linW, linb):
    x = x.astype(jnp.int32)
    src = edge_index[0].astype(jnp.int32)
    dst = edge_index[1].astype(jnp.int32)

    xp = jnp.concatenate([x, jnp.zeros((_NP - _N,), jnp.int32)])
    xp = xp.reshape(_NW, _NGCH, _GCH)
    dstw = dst.reshape(_NW, _EDW)
    srcr = jnp.concatenate([src, jnp.zeros((_EP - _E,), jnp.int32)])
    srcr = srcr.reshape(_NW, _NCHUNK, _ECH)
    pad_dst = _N + jnp.arange(_EP - _E, dtype=jnp.int32) % (_NP - _N)
    dstr = jnp.concatenate([dst, pad_dst])
    dstr = dstr.reshape(_NW, _NCHUNK, _ECH)
    batch2d = jnp.concatenate(
        [batch.astype(jnp.int32), jnp.full((_NP - _N,), _NGRAPH, jnp.int32)]
    ).reshape(_NP, 1)
    zeros_slab = jnp.zeros((_SLAB, _HID), jnp.float32)
    ones_nw = jnp.ones((_NW, 1), jnp.float32)

    h0, cnt = _sc_gather_deg(emb, xp, dstw)
    g1, dis = _tc_prep1(h0, W1, cnt, ones_nw)
    p1 = _sc_scatter(g1, srcr, dstr, zeros_slab)
    g2 = _tc_mid(p1[0], p1[1], g1, dis, b1.reshape(1, _HID), W2)
    p2 = _sc_scatter(g2, srcr, dstr, zeros_slab)
    out = _tc_final(p2[0], p2[1], g2, dis, b2.reshape(1, _HID), batch2d,
                    linW, linb.reshape(1, _NCLS))
    return out


# speedup vs baseline: 1.5099x; 1.5087x over previous
import functools

import jax
import jax.numpy as jnp
from jax import lax
from jax.experimental import pallas as pl
from jax.experimental.pallas import tpu as pltpu
from jax.experimental.pallas import tpu_sc as plsc

_N = 10000
_E = 320000
_EMB = 128
_HID = 128
_NCLS = 10
_NGRAPH = 64

_NC = 2
_NS = 16
_NW = _NC * _NS

_NP = 10240
_RPW = _NP // _NW
_GCH = 80
_NGCH = _RPW // _GCH
_EP = 323584
_ET = _EP // _NW
_ECH = 128
_NCHUNK = _ET // _ECH
_EDW = _E // _NW
_SLAB = _NP // _NS

_BR = 1024
_NBLK = _NP // _BR

_sc_mesh = plsc.VectorSubcoreMesh(core_axis_name="c", subcore_axis_name="s")


@functools.partial(
    pl.kernel,
    out_type=[
        jax.ShapeDtypeStruct((_NP, _EMB), jnp.float32),
        jax.ShapeDtypeStruct((_NW, _NP), jnp.float32),
    ],
    mesh=_sc_mesh,
    scratch_types=[
        pltpu.VMEM((_NGCH, _GCH), jnp.int32),
        pltpu.VMEM((_GCH, _EMB), jnp.float32),
        pltpu.VMEM((_EDW,), jnp.int32),
        pltpu.VMEM((_NP,), jnp.float32),
        pltpu.SemaphoreType.DMA,
    ],
    compiler_params=pltpu.CompilerParams(needs_layout_passes=False),
)
def _sc_gather_deg(emb_hbm, xp_hbm, dstw_hbm, h0_hbm, cnt_hbm,
                   xbuf, rowbuf, dstbuf, cntbuf, sem):
    cid = lax.axis_index("c")
    sid = lax.axis_index("s")
    wid = sid * _NC + cid

    pltpu.sync_copy(xp_hbm.at[wid], xbuf)
    for j in range(_NGCH):
        pltpu.async_copy(emb_hbm.at[xbuf.at[j]], rowbuf, sem).wait()
        pltpu.sync_copy(rowbuf, h0_hbm.at[pl.ds(wid * _RPW + j * _GCH, _GCH)])

    pltpu.sync_copy(dstw_hbm.at[wid], dstbuf)
    zero16 = jnp.zeros((16,), jnp.float32)

    def zbody(i, carry):
        cntbuf[pl.ds(i * 16, 16)] = zero16
        return carry

    lax.fori_loop(0, _NP // 16, zbody, 0)

    ones16 = jnp.full((16,), 1.0, jnp.float32)

    def cbody(i, carry):
        idxv = dstbuf[pl.ds(i * 16, 16)]
        plsc.addupdate_scatter(cntbuf, [idxv], ones16)
        return carry

    lax.fori_loop(0, _EDW // 16, cbody, 0)
    pltpu.sync_copy(cntbuf, cnt_hbm.at[wid])


@functools.partial(
    pl.kernel,
    out_type=jax.ShapeDtypeStruct((_NC, _NP, _HID), jnp.float32),
    mesh=_sc_mesh,
    scratch_types=[
        pltpu.VMEM((_NCHUNK, _ECH), jnp.int32),
        pltpu.VMEM((_NCHUNK, _ECH), jnp.int32),
        pltpu.VMEM((_ECH, _HID), jnp.float32),
        pltpu.VMEM_SHARED((_NP, _HID), jnp.float32),
        pltpu.SemaphoreType.DMA,
    ],
)
def _sc_scatter(g_hbm, srcr_hbm, dstr_hbm, zeros_hbm, out_hbm,
                sbuf, dbuf, rowbuf, acc, sem):
    cid = lax.axis_index("c")
    sid = lax.axis_index("s")
    wid = sid * _NC + cid

    pltpu.sync_copy(zeros_hbm, acc.at[pl.ds(sid * _SLAB, _SLAB)])
    pltpu.sync_copy(srcr_hbm.at[wid], sbuf)
    pltpu.sync_copy(dstr_hbm.at[wid], dbuf)
    plsc.subcore_barrier()

    def body(j, carry):
        pltpu.async_copy(g_hbm.at[sbuf.at[j]], rowbuf, sem).wait()
        pltpu.sync_copy(rowbuf, acc.at[dbuf.at[j]], add=True)
        return carry

    lax.fori_loop(0, _NCHUNK, body, 0)
    plsc.subcore_barrier()
    pltpu.sync_copy(acc.at[pl.ds(sid * _SLAB, _SLAB)],
                    out_hbm.at[cid, pl.ds(sid * _SLAB, _SLAB)])


def _tc_prep1_body(h0_ref, w_ref, cnt_ref, ones_ref, g_ref, dis_ref):
    deg = lax.dot_general(cnt_ref[...], ones_ref[...], (((0,), (0,)), ((), ())),
                          preferred_element_type=jnp.float32)
    dis = lax.rsqrt(deg + 1.0)
    g_ref[...] = dis * jnp.dot(h0_ref[...], w_ref[...],
                               preferred_element_type=jnp.float32)
    dis_ref[...] = dis


_tc_prep1 = pl.pallas_call(
    _tc_prep1_body,
    grid=(_NBLK,),
    in_specs=[
        pl.BlockSpec((_BR, _EMB), lambda i: (i, 0)),
        pl.BlockSpec((_EMB, _HID), lambda i: (0, 0)),
        pl.BlockSpec((_NW, _BR), lambda i: (0, i)),
        pl.BlockSpec((_NW, 1), lambda i: (0, 0)),
    ],
    out_specs=[
        pl.BlockSpec((_BR, _HID), lambda i: (i, 0)),
        pl.BlockSpec((_BR, 1), lambda i: (i, 0)),
    ],
    out_shape=[
        jax.ShapeDtypeStruct((_NP, _HID), jnp.float32),
        jax.ShapeDtypeStruct((_NP, 1), jnp.float32),
    ],
)


def _tc_mid_body(p0_ref, p1_ref, g_ref, dis_ref, b_ref, w_ref, g2_ref):
    dis = dis_ref[...]
    h = jnp.maximum(dis * (p0_ref[...] + p1_ref[...] + g_ref[...]) + b_ref[...],
                    0.0)
    g2_ref[...] = dis * jnp.dot(h, w_ref[...],
                                preferred_element_type=jnp.float32)


_tc_mid = pl.pallas_call(
    _tc_mid_body,
    grid=(_NBLK,),
    in_specs=[
        pl.BlockSpec((_BR, _HID), lambda i: (i, 0)),
        pl.BlockSpec((_BR, _HID), lambda i: (i, 0)),
        pl.BlockSpec((_BR, _HID), lambda i: (i, 0)),
        pl.BlockSpec((_BR, 1), lambda i: (i, 0)),
        pl.BlockSpec((1, _HID), lambda i: (0, 0)),
        pl.BlockSpec((_HID, _HID), lambda i: (0, 0)),
    ],
    out_specs=pl.BlockSpec((_BR, _HID), lambda i: (i, 0)),
    out_shape=jax.ShapeDtypeStruct((_NP, _HID), jnp.float32),
)


def _tc_final_body(p0_ref, p1_ref, g_ref, dis_ref, b_ref, batch_ref,
                   linw_ref, linb_ref, out_ref, pacc, cacc):
    i = pl.program_id(0)
    dis = dis_ref[...]
    h = jnp.maximum(dis * (p0_ref[...] + p1_ref[...] + g_ref[...]) + b_ref[...],
                    0.0)
    gids = lax.broadcasted_iota(jnp.int32, (1, _NGRAPH), 1)
    onehot = (batch_ref[...] == gids).astype(jnp.float32)
    ps = lax.dot_general(onehot, h, (((0,), (0,)), ((), ())),
                         preferred_element_type=jnp.float32)
    cs = lax.dot_general(onehot, jnp.ones((_BR, 1), jnp.float32),
                         (((0,), (0,)), ((), ())),
                         preferred_element_type=jnp.float32)

    @pl.when(i == 0)
    def _():
        pacc[...] = ps
        cacc[...] = cs

    @pl.when(i != 0)
    def _():
        pacc[...] += ps
        cacc[...] += cs

    @pl.when(i == _NBLK - 1)
    def _():
        pooled = pacc[...] / jnp.maximum(cacc[...], 1.0)
        out_ref[...] = (jnp.dot(pooled, linw_ref[...],
                                preferred_element_type=jnp.float32)
                        + linb_ref[...])


_tc_final = pl.pallas_call(
    _tc_final_body,
    grid=(_NBLK,),
    in_specs=[
        pl.BlockSpec((_BR, _HID), lambda i: (i, 0)),
        pl.BlockSpec((_BR, _HID), lambda i: (i, 0)),
        pl.BlockSpec((_BR, _HID), lambda i: (i, 0)),
        pl.BlockSpec((_BR, 1), lambda i: (i, 0)),
        pl.BlockSpec((1, _HID), lambda i: (0, 0)),
        pl.BlockSpec((_BR, 1), lambda i: (i, 0)),
        pl.BlockSpec((_HID, _NCLS), lambda i: (0, 0)),
        pl.BlockSpec((1, _NCLS), lambda i: (0, 0)),
    ],
    out_specs=pl.BlockSpec((_NGRAPH, _NCLS), lambda i: (0, 0)),
    out_shape=jax.ShapeDtypeStruct((_NGRAPH, _NCLS), jnp.float32),
    scratch_shapes=[
        pltpu.VMEM((_NGRAPH, _HID), jnp.float32),
        pltpu.VMEM((_NGRAPH, 1), jnp.float32),
    ],
)


def kernel(x, edge_index, batch, emb, W1, b1, W2, b2, linW, linb):
    x = x.astype(jnp.int32)
    src = edge_index[0].astype(jnp.int32)
    dst = edge_index[1].astype(jnp.int32)

    xp = jnp.concatenate([x, jnp.zeros((_NP - _N,), jnp.int32)])
    xp = xp.reshape(_NW, _NGCH, _GCH)
    dstw = dst.reshape(_NW, _EDW)
    srcr = jnp.concatenate([src, jnp.zeros((_EP - _E,), jnp.int32)])
    srcr = srcr.reshape(_NW, _NCHUNK, _ECH)
    pad_dst = _N + jnp.arange(_EP - _E, dtype=jnp.int32) % (_NP - _N)
    dstr = jnp.concatenate([dst, pad_dst])
    dstr = dstr.reshape(_NW, _NCHUNK, _ECH)
    batch2d = jnp.concatenate(
        [batch.astype(jnp.int32), jnp.full((_NP - _N,), _NGRAPH, jnp.int32)]
    ).reshape(_NP, 1)
    zeros_slab = jnp.zeros((_SLAB, _HID), jnp.float32)
    ones_nw = jnp.ones((_NW, 1), jnp.float32)

    h0, cnt = _sc_gather_deg(emb, xp, dstw)
    g1, dis = _tc_prep1(h0, W1, cnt, ones_nw)
    p1 = _sc_scatter(g1, srcr, dstr, zeros_slab)
    g2 = _tc_mid(p1[0], p1[1], g1, dis, b1.reshape(1, _HID), W2)
    p2 = _sc_scatter(g2, srcr, dstr, zeros_slab)
    out = _tc_final(p2[0], p2[1], g2, dis, b2.reshape(1, _HID), batch2d,
                    linW, linb.reshape(1, _NCLS))
    return out


# speedup vs baseline: 1.7859x; 1.1828x over previous
import functools

import jax
import jax.numpy as jnp
from jax import lax
from jax.experimental import pallas as pl
from jax.experimental.pallas import tpu as pltpu
from jax.experimental.pallas import tpu_sc as plsc

_N = 10000
_E = 320000
_EMB = 128
_HID = 128
_NCLS = 10
_NGRAPH = 64

_NC = 2
_NS = 16
_NW = _NC * _NS

_NP = 10240
_RPW = _NP // _NW
_GCH = 80
_NGCH = _RPW // _GCH
_EP = 323584
_ET = _EP // _NW
_ECH = 128
_NCHUNK = _ET // _ECH
_STAGE = 40
_EDW = _E // _NW
_SLAB = _NP // _NS

_BR = 1024
_NBLK = _NP // _BR

_sc_mesh = plsc.VectorSubcoreMesh(core_axis_name="c", subcore_axis_name="s")


@functools.partial(
    pl.kernel,
    out_type=[
        jax.ShapeDtypeStruct((_NP, _EMB), jnp.float32),
        jax.ShapeDtypeStruct((_NW, _NP), jnp.float32),
    ],
    mesh=_sc_mesh,
    scratch_types=[
        pltpu.VMEM((_NGCH, _GCH), jnp.int32),
        pltpu.VMEM((_GCH, _EMB), jnp.float32),
        pltpu.VMEM((_EDW,), jnp.int32),
        pltpu.VMEM((_NP,), jnp.float32),
        pltpu.SemaphoreType.DMA,
    ],
    compiler_params=pltpu.CompilerParams(needs_layout_passes=False),
)
def _sc_gather_deg(emb_hbm, xp_hbm, dstw_hbm, h0_hbm, cnt_hbm,
                   xbuf, rowbuf, dstbuf, cntbuf, sem):
    cid = lax.axis_index("c")
    sid = lax.axis_index("s")
    wid = sid * _NC + cid

    pltpu.sync_copy(xp_hbm.at[wid], xbuf)
    for j in range(_NGCH):
        pltpu.async_copy(emb_hbm.at[xbuf.at[j]], rowbuf, sem).wait()
        pltpu.sync_copy(rowbuf, h0_hbm.at[pl.ds(wid * _RPW + j * _GCH, _GCH)])

    pltpu.sync_copy(dstw_hbm.at[wid], dstbuf)
    zero16 = jnp.zeros((16,), jnp.float32)

    def zbody(i, carry):
        cntbuf[pl.ds(i * 16, 16)] = zero16
        return carry

    lax.fori_loop(0, _NP // 16, zbody, 0)

    ones16 = jnp.full((16,), 1.0, jnp.float32)

    def cbody(i, carry):
        idxv = dstbuf[pl.ds(i * 16, 16)]
        plsc.addupdate_scatter(cntbuf, [idxv], ones16)
        return carry

    lax.fori_loop(0, _EDW // 16, cbody, 0)
    pltpu.sync_copy(cntbuf, cnt_hbm.at[wid])


@functools.partial(
    pl.kernel,
    out_type=jax.ShapeDtypeStruct((_NC, _NP, _HID), jnp.float32),
    mesh=_sc_mesh,
    scratch_types=[
        pltpu.VMEM((_STAGE, _ECH), jnp.int32),
        pltpu.VMEM((_STAGE, _ECH), jnp.int32),
        pltpu.VMEM((_ECH, _HID), jnp.float32),
        pltpu.VMEM((_ECH, _HID), jnp.float32),
        pltpu.VMEM_SHARED((_NP, _HID), jnp.float32),
        pltpu.SemaphoreType.DMA,
        pltpu.SemaphoreType.DMA,
    ],
)
def _sc_scatter(g_hbm, srcr_hbm, dstr_hbm, zeros_hbm, out_hbm,
                sbuf, dbuf, rb0, rb1, acc, g0, g1):
    cid = lax.axis_index("c")
    sid = lax.axis_index("s")
    wid = sid * _NC + cid

    pltpu.sync_copy(zeros_hbm, acc.at[pl.ds(sid * _SLAB, _SLAB)])
    plsc.subcore_barrier()

    for h, nch in ((0, _STAGE), (1, _NCHUNK - _STAGE)):
        npair = nch // 2
        pltpu.sync_copy(srcr_hbm.at[wid, pl.ds(h * _STAGE, nch)],
                        sbuf.at[pl.ds(0, nch)])
        pltpu.sync_copy(dstr_hbm.at[wid, pl.ds(h * _STAGE, nch)],
                        dbuf.at[pl.ds(0, nch)])
        pltpu.async_copy(g_hbm.at[sbuf.at[0]], rb0, g0)

        def body(p, carry):
            j = 2 * p
            pltpu.async_copy(g_hbm.at[sbuf.at[j + 1]], rb1, g1)
            pltpu.make_async_copy(g_hbm.at[sbuf.at[0]], rb0, g0).wait()
            pltpu.sync_copy(rb0, acc.at[dbuf.at[j]], add=True)

            @pl.when(p + 1 < npair)
            def _():
                pltpu.async_copy(g_hbm.at[sbuf.at[j + 2]], rb0, g0)

            pltpu.make_async_copy(g_hbm.at[sbuf.at[1]], rb1, g1).wait()
            pltpu.sync_copy(rb1, acc.at[dbuf.at[j + 1]], add=True)
            return carry

        lax.fori_loop(0, npair, body, 0)

        if nch % 2:
            pltpu.async_copy(g_hbm.at[sbuf.at[nch - 1]], rb0, g0)
            pltpu.make_async_copy(g_hbm.at[sbuf.at[0]], rb0, g0).wait()
            pltpu.sync_copy(rb0, acc.at[dbuf.at[nch - 1]], add=True)
    plsc.subcore_barrier()
    pltpu.sync_copy(acc.at[pl.ds(sid * _SLAB, _SLAB)],
                    out_hbm.at[cid, pl.ds(sid * _SLAB, _SLAB)])


def _tc_prep1_body(h0_ref, w_ref, cnt_ref, ones_ref, g_ref, dis_ref):
    deg = lax.dot_general(cnt_ref[...], ones_ref[...], (((0,), (0,)), ((), ())),
                          preferred_element_type=jnp.float32)
    dis = lax.rsqrt(deg + 1.0)
    g_ref[...] = dis * jnp.dot(h0_ref[...], w_ref[...],
                               preferred_element_type=jnp.float32)
    dis_ref[...] = dis


_tc_prep1 = pl.pallas_call(
    _tc_prep1_body,
    grid=(_NBLK,),
    in_specs=[
        pl.BlockSpec((_BR, _EMB), lambda i: (i, 0)),
        pl.BlockSpec((_EMB, _HID), lambda i: (0, 0)),
        pl.BlockSpec((_NW, _BR), lambda i: (0, i)),
        pl.BlockSpec((_NW, 1), lambda i: (0, 0)),
    ],
    out_specs=[
        pl.BlockSpec((_BR, _HID), lambda i: (i, 0)),
        pl.BlockSpec((_BR, 1), lambda i: (i, 0)),
    ],
    out_shape=[
        jax.ShapeDtypeStruct((_NP, _HID), jnp.float32),
        jax.ShapeDtypeStruct((_NP, 1), jnp.float32),
    ],
)


def _tc_mid_body(p0_ref, p1_ref, g_ref, dis_ref, b_ref, w_ref, g2_ref):
    dis = dis_ref[...]
    h = jnp.maximum(dis * (p0_ref[...] + p1_ref[...] + g_ref[...]) + b_ref[...],
                    0.0)
    g2_ref[...] = dis * jnp.dot(h, w_ref[...],
                                preferred_element_type=jnp.float32)


_tc_mid = pl.pallas_call(
    _tc_mid_body,
    grid=(_NBLK,),
    in_specs=[
        pl.BlockSpec((_BR, _HID), lambda i: (i, 0)),
        pl.BlockSpec((_BR, _HID), lambda i: (i, 0)),
        pl.BlockSpec((_BR, _HID), lambda i: (i, 0)),
        pl.BlockSpec((_BR, 1), lambda i: (i, 0)),
        pl.BlockSpec((1, _HID), lambda i: (0, 0)),
        pl.BlockSpec((_HID, _HID), lambda i: (0, 0)),
    ],
    out_specs=pl.BlockSpec((_BR, _HID), lambda i: (i, 0)),
    out_shape=jax.ShapeDtypeStruct((_NP, _HID), jnp.float32),
)


def _tc_final_body(p0_ref, p1_ref, g_ref, dis_ref, b_ref, batch_ref,
                   linw_ref, linb_ref, out_ref, pacc, cacc):
    i = pl.program_id(0)
    dis = dis_ref[...]
    h = jnp.maximum(dis * (p0_ref[...] + p1_ref[...] + g_ref[...]) + b_ref[...],
                    0.0)
    gids = lax.broadcasted_iota(jnp.int32, (1, _NGRAPH), 1)
    onehot = (batch_ref[...] == gids).astype(jnp.float32)
    ps = lax.dot_general(onehot, h, (((0,), (0,)), ((), ())),
                         preferred_element_type=jnp.float32)
    cs = lax.dot_general(onehot, jnp.ones((_BR, 1), jnp.float32),
                         (((0,), (0,)), ((), ())),
                         preferred_element_type=jnp.float32)

    @pl.when(i == 0)
    def _():
        pacc[...] = ps
        cacc[...] = cs

    @pl.when(i != 0)
    def _():
        pacc[...] += ps
        cacc[...] += cs

    @pl.when(i == _NBLK - 1)
    def _():
        pooled = pacc[...] / jnp.maximum(cacc[...], 1.0)
        out_ref[...] = (jnp.dot(pooled, linw_ref[...],
                                preferred_element_type=jnp.float32)
                        + linb_ref[...])


_tc_final = pl.pallas_call(
    _tc_final_body,
    grid=(_NBLK,),
    in_specs=[
        pl.BlockSpec((_BR, _HID), lambda i: (i, 0)),
        pl.BlockSpec((_BR, _HID), lambda i: (i, 0)),
        pl.BlockSpec((_BR, _HID), lambda i: (i, 0)),
        pl.BlockSpec((_BR, 1), lambda i: (i, 0)),
        pl.BlockSpec((1, _HID), lambda i: (0, 0)),
        pl.BlockSpec((_BR, 1), lambda i: (i, 0)),
        pl.BlockSpec((_HID, _NCLS), lambda i: (0, 0)),
        pl.BlockSpec((1, _NCLS), lambda i: (0, 0)),
    ],
    out_specs=pl.BlockSpec((_NGRAPH, _NCLS), lambda i: (0, 0)),
    out_shape=jax.ShapeDtypeStruct((_NGRAPH, _NCLS), jnp.float32),
    scratch_shapes=[
        pltpu.VMEM((_NGRAPH, _HID), jnp.float32),
        pltpu.VMEM((_NGRAPH, 1), jnp.float32),
    ],
)


def kernel(x, edge_index, batch, emb, W1, b1, W2, b2, linW, linb):
    x = x.astype(jnp.int32)
    src = edge_index[0].astype(jnp.int32)
    dst = edge_index[1].astype(jnp.int32)

    xp = jnp.concatenate([x, jnp.zeros((_NP - _N,), jnp.int32)])
    xp = xp.reshape(_NW, _NGCH, _GCH)
    dstw = dst.reshape(_NW, _EDW)
    srcr = jnp.concatenate([src, jnp.zeros((_EP - _E,), jnp.int32)])
    srcr = srcr.reshape(_NW, _NCHUNK, _ECH)
    pad_dst = _N + jnp.arange(_EP - _E, dtype=jnp.int32) % (_NP - _N)
    dstr = jnp.concatenate([dst, pad_dst])
    dstr = dstr.reshape(_NW, _NCHUNK, _ECH)
    batch2d = jnp.concatenate(
        [batch.astype(jnp.int32), jnp.full((_NP - _N,), _NGRAPH, jnp.int32)]
    ).reshape(_NP, 1)
    zeros_slab = jnp.zeros((_SLAB, _HID), jnp.float32)
    ones_nw = jnp.ones((_NW, 1), jnp.float32)

    h0, cnt = _sc_gather_deg(emb, xp, dstw)
    g1, dis = _tc_prep1(h0, W1, cnt, ones_nw)
    p1 = _sc_scatter(g1, srcr, dstr, zeros_slab)
    g2 = _tc_mid(p1[0], p1[1], g1, dis, b1.reshape(1, _HID), W2)
    p2 = _sc_scatter(g2, srcr, dstr, zeros_slab)
    out = _tc_final(p2[0], p2[1], g2, dis, b2.reshape(1, _HID), batch2d,
                    linW, linb.reshape(1, _NCLS))
    return out
